# baseline (speedup 1.0000x reference)
import functools

import jax
import jax.numpy as jnp
from jax.experimental import pallas as pl
from jax.experimental.pallas import tpu as pltpu

_NEG = float(jnp.finfo(jnp.float32).min)
_CC = 256
_H = 128
_W = 128
_HW = _H * _W


def _obj_pool_body(b_ref, w_ref, f_ref, out_ref, acc_ref):
    ci = pl.program_id(1)
    ncs = pl.num_programs(1)
    fb = f_ref[0]
    w = w_ref[...]
    part = jnp.dot(w, fb, preferred_element_type=jnp.float32)

    @pl.when(ci == 0)
    def _init():
        acc_ref[...] = part

    @pl.when(ci != 0)
    def _acc():
        acc_ref[...] = acc_ref[...] + part

    @pl.when(ci == ncs - 1)
    def _fin():
        x = (acc_ref[...] + b_ref[0]).reshape(_H, _W)
        ninf_row = jnp.full((1, _W), _NEG, jnp.float32)
        up = jnp.concatenate([x[1:], ninf_row], axis=0)
        dn = jnp.concatenate([ninf_row, x[:-1]], axis=0)
        v = jnp.maximum(jnp.maximum(x, up), dn)
        ninf_col = jnp.full((_H, 1), _NEG, jnp.float32)
        lf = jnp.concatenate([v[:, 1:], ninf_col], axis=1)
        rt = jnp.concatenate([ninf_col, v[:, :-1]], axis=1)
        p = jnp.maximum(jnp.maximum(v, lf), rt)
        out_ref[0] = jnp.where(p == x, x, _NEG)


def _masked_obj(feat, W, b):
    B, C, H, Wd = feat.shape
    fr = feat.reshape(B, C, H * Wd)
    grid = (B, C // _CC)
    return pl.pallas_call(
        _obj_pool_body,
        grid=grid,
        in_specs=[
            pl.BlockSpec(memory_space=pltpu.SMEM),
            pl.BlockSpec((1, _CC), lambda bi, ci: (0, ci)),
            pl.BlockSpec((1, _CC, _HW), lambda bi, ci: (bi, ci, 0)),
        ],
        out_specs=pl.BlockSpec((1, _H, _W), lambda bi, ci: (bi, 0, 0)),
        out_shape=jax.ShapeDtypeStruct((B, _H, _W), jnp.float32),
        scratch_shapes=[pltpu.VMEM((1, _HW), jnp.float32)],
        compiler_params=pltpu.CompilerParams(
            dimension_semantics=("parallel", "arbitrary"),
        ),
    )(b, W, fr)


def kernel(feat, W, b):
    masked = _masked_obj(feat, W, b).reshape(feat.shape[0], -1)
    sel_scores, top_inds = jax.lax.top_k(masked, 1000)
    return sel_scores, top_inds
